# baseline (speedup 1.0000x reference)
import functools

import jax
import jax.numpy as jnp
from jax import lax
from jax.experimental import pallas as pl
from jax.experimental.pallas import tpu as pltpu
from jax.experimental.pallas import tpu_sc as plsc

NC = 2
NS = 16
NW = NC * NS
D = 128
BPC = 256
IB = 128


def _phase(idx_hbm, table_hbm, out_hbm, *, bag, total, wid, slots):
    ns = BPC // IB
    n = -(-total // (NW * BPC))
    n = -(-n // 3) * 3
    tile0 = wid * n * BPC
    last = total - BPC

    def start_of(c):
        return jnp.minimum(tile0 + c * BPC, last)

    def stage(c, idxb, sem):
        s0 = start_of(c)
        for j in range(bag):
            pltpu.async_copy(idx_hbm.at[pl.ds(j * total + s0, BPC)],
                             idxb.at[pl.ds(j * BPC, BPC)], sem)

    def stage_wait(c, idxb, sem):
        s0 = start_of(c)
        for j in range(bag):
            pltpu.make_async_copy(idx_hbm.at[pl.ds(j * total + s0, BPC)],
                                  idxb.at[pl.ds(j * BPC, BPC)], sem).wait()

    def fire_base(idxb, rowsb, sem):
        for b in range(ns):
            pltpu.async_copy(table_hbm.at[idxb.at[pl.ds(b * IB, IB)]],
                             rowsb.at[pl.ds(b * IB, IB)], sem)

    def drain_base(idxb, rowsb, sem):
        for b in range(ns):
            pltpu.make_async_copy(table_hbm.at[idxb.at[pl.ds(b * IB, IB)]],
                                  rowsb.at[pl.ds(b * IB, IB)], sem).wait()

    def fire_add(idxb, rowsb, sem):
        for j in range(1, bag):
            for b in range(ns):
                pltpu.async_copy(
                    table_hbm.at[idxb.at[pl.ds(j * BPC + b * IB, IB)]],
                    rowsb.at[pl.ds(b * IB, IB)], sem, add=True)

    def drain_add(idxb, rowsb, sem):
        for j in range(1, bag):
            for b in range(ns):
                pltpu.make_async_copy(
                    table_hbm.at[idxb.at[pl.ds(j * BPC + b * IB, IB)]],
                    rowsb.at[pl.ds(b * IB, IB)], sem).wait()

    def put(c, rowsb, sem):
        pltpu.async_copy(rowsb.at[pl.ds(0, BPC)],
                         out_hbm.at[pl.ds(start_of(c), BPC)], sem)

    def put_wait(c, rowsb, sem):
        pltpu.make_async_copy(rowsb.at[pl.ds(0, BPC)],
                              out_hbm.at[pl.ds(start_of(c), BPC)], sem).wait()

    for k in range(3):
        stage(k, slots[k][0], slots[k][3])
    stage_wait(0, slots[0][0], slots[0][3])
    fire_base(slots[0][0], slots[0][1], slots[0][2])

    def step(c, X, Y):
        idxX, rowsX, sgX, siX, soX = X
        idxY, rowsY, sgY, siY, soY = Y
        drain_base(idxX, rowsX, sgX)
        fire_add(idxX, rowsX, sgX)

        @pl.when(c + 1 < n)
        def _():
            stage_wait(c + 1, idxY, siY)

            @pl.when(c >= 2)
            def _():
                put_wait(c - 2, rowsY, soY)

            fire_base(idxY, rowsY, sgY)

        drain_add(idxX, rowsX, sgX)

        @pl.when(c + 3 < n)
        def _():
            stage(c + 3, idxX, siX)

        put(c, rowsX, soX)

    def outer(cc, _):
        c0 = 3 * cc
        for k in range(3):
            step(c0 + k, slots[k], slots[(k + 1) % 3])
        return 0

    lax.fori_loop(0, n // 3, outer, 0, unroll=False)
    for c in (n - 3, n - 2, n - 1):
        put_wait(c, slots[c % 3][1], slots[c % 3][4])


def _sc_kernel(nv, ne):
    mesh = plsc.VectorSubcoreMesh(core_axis_name="c", subcore_axis_name="s")

    @functools.partial(
        pl.kernel,
        out_type=(
            jax.ShapeDtypeStruct((nv, D), jnp.float32),
            jax.ShapeDtypeStruct((ne, D), jnp.float32),
        ),
        mesh=mesh,
        compiler_params=pltpu.CompilerParams(use_tc_tiling_on_sc=True),
        scratch_types=[
            pltpu.VMEM((4 * BPC,), jnp.int32),
            pltpu.VMEM((4 * BPC,), jnp.int32),
            pltpu.VMEM((4 * BPC,), jnp.int32),
            pltpu.VMEM((BPC, D), jnp.float32),
            pltpu.VMEM((BPC, D), jnp.float32),
            pltpu.VMEM((BPC, D), jnp.float32),
            pltpu.SemaphoreType.DMA,
            pltpu.SemaphoreType.DMA,
            pltpu.SemaphoreType.DMA,
            pltpu.SemaphoreType.DMA,
            pltpu.SemaphoreType.DMA,
            pltpu.SemaphoreType.DMA,
            pltpu.SemaphoreType.DMA,
            pltpu.SemaphoreType.DMA,
            pltpu.SemaphoreType.DMA,
            pltpu.VMEM_SHARED((1000, D), jnp.float32),
        ],
    )
    def k(vidx_hbm, eidx_hbm, node_hbm, edge_hbm, vout_hbm, eout_hbm,
          idx0, idx1, idx2, rows0, rows1, rows2,
          sg0, sg1, sg2, si0, si1, si2, so0, so1, so2, etab_sp):
        wid = lax.axis_index("s") * NC + lax.axis_index("c")
        sid = lax.axis_index("s")
        slots = ((idx0, rows0, sg0, si0, so0),
                 (idx1, rows1, sg1, si1, so1),
                 (idx2, rows2, sg2, si2, so2))

        @pl.when(sid < 7)
        def _():
            pltpu.sync_copy(edge_hbm.at[pl.ds(sid * 128, 128)],
                            etab_sp.at[pl.ds(sid * 128, 128)])

        @pl.when(sid == 7)
        def _():
            pltpu.sync_copy(edge_hbm.at[pl.ds(896, 104)],
                            etab_sp.at[pl.ds(896, 104)])

        _phase(vidx_hbm, node_hbm, vout_hbm, bag=4, total=nv, wid=wid,
               slots=slots)
        plsc.subcore_barrier()
        _phase(eidx_hbm, etab_sp, eout_hbm, bag=2, total=ne, wid=wid,
               slots=slots)

    return k


def kernel(V, E, node_table, edge_table):
    nv, bag_v = V.shape
    ne, bag_e = E.shape
    assert bag_v == 4 and bag_e == 2 and nv % 8 == 0 and ne % 8 == 0
    vflat = jnp.concatenate([V[:, j].astype(jnp.int32) for j in range(bag_v)])
    eflat = jnp.concatenate([E[:, j].astype(jnp.int32) for j in range(bag_e)])
    return _sc_kernel(nv, ne)(vflat, eflat, node_table, edge_table)


# speedup vs baseline: 1.0322x; 1.0322x over previous
import functools

import jax
import jax.numpy as jnp
from jax import lax
from jax.experimental import pallas as pl
from jax.experimental.pallas import tpu as pltpu
from jax.experimental.pallas import tpu_sc as plsc

NC = 2
NS = 16
NW = NC * NS
D = 128
BPC = 256
IB = 128


def _phase(idx_hbm, table_hbm, out_hbm, *, bag, total, bpc, wid, slots):
    ns = -(-bpc // IB)
    n = -(-total // (NW * bpc))
    n = -(-n // 3) * 3
    tile0 = wid * n * bpc
    last = total - bpc
    K = 16

    def start_of(c):
        s0 = tile0 + c * bpc
        over = s0 - last
        k = jnp.where(over > 0, ((over + bpc - 1) // bpc) % K, 0)
        return jnp.where(over > 0, last - k * bpc, s0)

    def stage(c, idxb, sem):
        s0 = start_of(c)
        for j in range(bag):
            pltpu.async_copy(idx_hbm.at[pl.ds(j * total + s0, bpc)],
                             idxb.at[pl.ds(j * bpc, bpc)], sem)

    def stage_wait(c, idxb, sem):
        s0 = start_of(c)
        for j in range(bag):
            pltpu.make_async_copy(idx_hbm.at[pl.ds(j * total + s0, bpc)],
                                  idxb.at[pl.ds(j * bpc, bpc)], sem).wait()

    def fire_base(idxb, rowsb, sem):
        for b in range(ns):
            pltpu.async_copy(table_hbm.at[idxb.at[pl.ds(b * IB, IB)]],
                             rowsb.at[pl.ds(b * IB, IB)], sem)

    def drain_base(idxb, rowsb, sem):
        for b in range(ns):
            pltpu.make_async_copy(table_hbm.at[idxb.at[pl.ds(b * IB, IB)]],
                                  rowsb.at[pl.ds(b * IB, IB)], sem).wait()

    def fire_add(idxb, rowsb, sem):
        for j in range(1, bag):
            for b in range(ns):
                pltpu.async_copy(
                    table_hbm.at[idxb.at[pl.ds(j * bpc + b * IB, IB)]],
                    rowsb.at[pl.ds(b * IB, IB)], sem, add=True)

    def drain_add(idxb, rowsb, sem):
        for j in range(1, bag):
            for b in range(ns):
                pltpu.make_async_copy(
                    table_hbm.at[idxb.at[pl.ds(j * bpc + b * IB, IB)]],
                    rowsb.at[pl.ds(b * IB, IB)], sem).wait()

    def put(c, rowsb, sem):
        pltpu.async_copy(rowsb.at[pl.ds(0, bpc)],
                         out_hbm.at[pl.ds(start_of(c), bpc)], sem)

    def put_wait(c, rowsb, sem):
        pltpu.make_async_copy(rowsb.at[pl.ds(0, bpc)],
                              out_hbm.at[pl.ds(start_of(c), bpc)], sem).wait()

    for k in range(3):
        stage(k, slots[k][0], slots[k][3])
    stage_wait(0, slots[0][0], slots[0][3])
    fire_base(slots[0][0], slots[0][1], slots[0][2])

    def step(c, X, Y):
        idxX, rowsX, sgX, siX, soX = X
        idxY, rowsY, sgY, siY, soY = Y
        drain_base(idxX, rowsX, sgX)
        fire_add(idxX, rowsX, sgX)

        @pl.when(c + 1 < n)
        def _():
            stage_wait(c + 1, idxY, siY)

            @pl.when(c >= 2)
            def _():
                put_wait(c - 2, rowsY, soY)

            fire_base(idxY, rowsY, sgY)

        drain_add(idxX, rowsX, sgX)

        @pl.when(c + 3 < n)
        def _():
            stage(c + 3, idxX, siX)

        put(c, rowsX, soX)

    def outer(cc, _):
        c0 = 3 * cc
        for k in range(3):
            step(c0 + k, slots[k], slots[(k + 1) % 3])
        return 0

    lax.fori_loop(0, n // 3, outer, 0, unroll=False)
    for c in (n - 3, n - 2, n - 1):
        put_wait(c, slots[c % 3][1], slots[c % 3][4])


def _sc_kernel(nv, ne):
    mesh = plsc.VectorSubcoreMesh(core_axis_name="c", subcore_axis_name="s")

    @functools.partial(
        pl.kernel,
        out_type=(
            jax.ShapeDtypeStruct((nv, D), jnp.float32),
            jax.ShapeDtypeStruct((ne, D), jnp.float32),
        ),
        mesh=mesh,
        compiler_params=pltpu.CompilerParams(use_tc_tiling_on_sc=True),
        scratch_types=[
            pltpu.VMEM((4 * BPC,), jnp.int32),
            pltpu.VMEM((4 * BPC,), jnp.int32),
            pltpu.VMEM((4 * BPC,), jnp.int32),
            pltpu.VMEM((BPC, D), jnp.float32),
            pltpu.VMEM((BPC, D), jnp.float32),
            pltpu.VMEM((BPC, D), jnp.float32),
            pltpu.SemaphoreType.DMA,
            pltpu.SemaphoreType.DMA,
            pltpu.SemaphoreType.DMA,
            pltpu.SemaphoreType.DMA,
            pltpu.SemaphoreType.DMA,
            pltpu.SemaphoreType.DMA,
            pltpu.SemaphoreType.DMA,
            pltpu.SemaphoreType.DMA,
            pltpu.SemaphoreType.DMA,
            pltpu.VMEM_SHARED((1000, D), jnp.float32),
        ],
    )
    def k(vidx_hbm, eidx_hbm, node_hbm, edge_hbm, vout_hbm, eout_hbm,
          idx0, idx1, idx2, rows0, rows1, rows2,
          sg0, sg1, sg2, si0, si1, si2, so0, so1, so2, etab_sp):
        wid = lax.axis_index("s") * NC + lax.axis_index("c")
        sid = lax.axis_index("s")
        slots = ((idx0, rows0, sg0, si0, so0),
                 (idx1, rows1, sg1, si1, so1),
                 (idx2, rows2, sg2, si2, so2))

        @pl.when(sid < 7)
        def _():
            pltpu.sync_copy(edge_hbm.at[pl.ds(sid * 128, 128)],
                            etab_sp.at[pl.ds(sid * 128, 128)])

        @pl.when(sid == 7)
        def _():
            pltpu.sync_copy(edge_hbm.at[pl.ds(896, 104)],
                            etab_sp.at[pl.ds(896, 104)])

        _phase(vidx_hbm, node_hbm, vout_hbm, bag=4, total=nv, bpc=256,
               wid=wid, slots=slots)
        plsc.subcore_barrier()
        _phase(eidx_hbm, etab_sp, eout_hbm, bag=2, total=ne, bpc=256,
               wid=wid, slots=slots)

    return k


def kernel(V, E, node_table, edge_table):
    nv, bag_v = V.shape
    ne, bag_e = E.shape
    assert bag_v == 4 and bag_e == 2 and nv % 8 == 0 and ne % 8 == 0
    vflat = jnp.concatenate([V[:, j].astype(jnp.int32) for j in range(bag_v)])
    eflat = jnp.concatenate([E[:, j].astype(jnp.int32) for j in range(bag_e)])
    return _sc_kernel(nv, ne)(vflat, eflat, node_table, edge_table)


# speedup vs baseline: 1.0740x; 1.0406x over previous
import functools

import jax
import jax.numpy as jnp
from jax import lax
from jax.experimental import pallas as pl
from jax.experimental.pallas import tpu as pltpu
from jax.experimental.pallas import tpu_sc as plsc

NC = 2
NS = 16
NW = NC * NS
D = 128
BPC = 256
IB = 128


def _phase(idx_hbm, table_hbm, out_hbm, *, bag, total, bpc, wid, slots):
    ns = -(-bpc // IB)
    n = max(-(-total // (NW * bpc)), 3)
    tile0 = wid * n * bpc
    last = total - bpc
    K = 16

    def start_of(c):
        s0 = tile0 + c * bpc
        over = s0 - last
        k = jnp.where(over > 0, ((over + bpc - 1) // bpc) % K, 0)
        return jnp.where(over > 0, last - k * bpc, s0)

    def stage(c, idxb, sem):
        s0 = start_of(c)
        for j in range(bag):
            pltpu.async_copy(idx_hbm.at[pl.ds(j * total + s0, bpc)],
                             idxb.at[pl.ds(j * bpc, bpc)], sem)

    def stage_wait(c, idxb, sem):
        s0 = start_of(c)
        for j in range(bag):
            pltpu.make_async_copy(idx_hbm.at[pl.ds(j * total + s0, bpc)],
                                  idxb.at[pl.ds(j * bpc, bpc)], sem).wait()

    def fire_base(idxb, rowsb, sem):
        for b in range(ns):
            pltpu.async_copy(table_hbm.at[idxb.at[pl.ds(b * IB, IB)]],
                             rowsb.at[pl.ds(b * IB, IB)], sem)

    def drain_base(idxb, rowsb, sem):
        for b in range(ns):
            pltpu.make_async_copy(table_hbm.at[idxb.at[pl.ds(b * IB, IB)]],
                                  rowsb.at[pl.ds(b * IB, IB)], sem).wait()

    def fire_add(idxb, rowsb, sem):
        for j in range(1, bag):
            for b in range(ns):
                pltpu.async_copy(
                    table_hbm.at[idxb.at[pl.ds(j * bpc + b * IB, IB)]],
                    rowsb.at[pl.ds(b * IB, IB)], sem, add=True)

    def drain_add(idxb, rowsb, sem):
        for j in range(1, bag):
            for b in range(ns):
                pltpu.make_async_copy(
                    table_hbm.at[idxb.at[pl.ds(j * bpc + b * IB, IB)]],
                    rowsb.at[pl.ds(b * IB, IB)], sem).wait()

    def put(c, rowsb, sem):
        pltpu.async_copy(rowsb.at[pl.ds(0, bpc)],
                         out_hbm.at[pl.ds(start_of(c), bpc)], sem)

    def put_wait(c, rowsb, sem):
        pltpu.make_async_copy(rowsb.at[pl.ds(0, bpc)],
                              out_hbm.at[pl.ds(start_of(c), bpc)], sem).wait()

    for k in range(3):
        stage(k, slots[k][0], slots[k][3])
    stage_wait(0, slots[0][0], slots[0][3])
    fire_base(slots[0][0], slots[0][1], slots[0][2])

    def step(c, X, Y):
        idxX, rowsX, sgX, siX, soX = X
        idxY, rowsY, sgY, siY, soY = Y
        drain_base(idxX, rowsX, sgX)
        fire_add(idxX, rowsX, sgX)

        @pl.when(c + 1 < n)
        def _():
            stage_wait(c + 1, idxY, siY)

            @pl.when(c >= 2)
            def _():
                put_wait(c - 2, rowsY, soY)

            fire_base(idxY, rowsY, sgY)

        drain_add(idxX, rowsX, sgX)

        @pl.when(c + 3 < n)
        def _():
            stage(c + 3, idxX, siX)

        put(c, rowsX, soX)

    def outer(cc, _):
        c0 = 3 * cc
        for k in range(3):
            step(c0 + k, slots[k], slots[(k + 1) % 3])
        return 0

    lax.fori_loop(0, n // 3, outer, 0, unroll=False)
    for c in range(n - n % 3, n):
        step(c, slots[c % 3], slots[(c + 1) % 3])
    for c in (n - 3, n - 2, n - 1):
        put_wait(c, slots[c % 3][1], slots[c % 3][4])


def _sc_kernel(nv, ne):
    mesh = plsc.VectorSubcoreMesh(core_axis_name="c", subcore_axis_name="s")

    @functools.partial(
        pl.kernel,
        out_type=(
            jax.ShapeDtypeStruct((nv, D), jnp.float32),
            jax.ShapeDtypeStruct((ne, D), jnp.float32),
        ),
        mesh=mesh,
        compiler_params=pltpu.CompilerParams(use_tc_tiling_on_sc=True),
        scratch_types=[
            pltpu.VMEM((4 * BPC,), jnp.int32),
            pltpu.VMEM((4 * BPC,), jnp.int32),
            pltpu.VMEM((4 * BPC,), jnp.int32),
            pltpu.VMEM((BPC, D), jnp.float32),
            pltpu.VMEM((BPC, D), jnp.float32),
            pltpu.VMEM((BPC, D), jnp.float32),
            pltpu.SemaphoreType.DMA,
            pltpu.SemaphoreType.DMA,
            pltpu.SemaphoreType.DMA,
            pltpu.SemaphoreType.DMA,
            pltpu.SemaphoreType.DMA,
            pltpu.SemaphoreType.DMA,
            pltpu.SemaphoreType.DMA,
            pltpu.SemaphoreType.DMA,
            pltpu.SemaphoreType.DMA,
            pltpu.VMEM_SHARED((1000, D), jnp.float32),
        ],
    )
    def k(vidx_hbm, eidx_hbm, node_hbm, edge_hbm, vout_hbm, eout_hbm,
          idx0, idx1, idx2, rows0, rows1, rows2,
          sg0, sg1, sg2, si0, si1, si2, so0, so1, so2, etab_sp):
        wid = lax.axis_index("s") * NC + lax.axis_index("c")
        sid = lax.axis_index("s")
        slots = ((idx0, rows0, sg0, si0, so0),
                 (idx1, rows1, sg1, si1, so1),
                 (idx2, rows2, sg2, si2, so2))

        @pl.when(sid < 7)
        def _():
            pltpu.sync_copy(edge_hbm.at[pl.ds(sid * 128, 128)],
                            etab_sp.at[pl.ds(sid * 128, 128)])

        @pl.when(sid == 7)
        def _():
            pltpu.sync_copy(edge_hbm.at[pl.ds(896, 104)],
                            etab_sp.at[pl.ds(896, 104)])

        _phase(vidx_hbm, node_hbm, vout_hbm, bag=4, total=nv, bpc=256,
               wid=wid, slots=slots)
        plsc.subcore_barrier()
        _phase(eidx_hbm, etab_sp, eout_hbm, bag=2, total=ne, bpc=256,
               wid=wid, slots=slots)

    return k


def kernel(V, E, node_table, edge_table):
    nv, bag_v = V.shape
    ne, bag_e = E.shape
    assert bag_v == 4 and bag_e == 2 and nv % 8 == 0 and ne % 8 == 0
    vflat = jnp.concatenate([V[:, j].astype(jnp.int32) for j in range(bag_v)])
    eflat = jnp.concatenate([E[:, j].astype(jnp.int32) for j in range(bag_e)])
    return _sc_kernel(nv, ne)(vflat, eflat, node_table, edge_table)
